# Initial kernel scaffold; baseline (speedup 1.0000x reference)
import jax
import jax.numpy as jnp
from jax.experimental import pallas as pl


def kernel(inputs, embedding):
    raise NotImplementedError("write your pallas kernel here")


# speedup vs baseline: 3.5436x; 3.5436x over previous
import functools

import jax
import jax.numpy as jnp
from jax.experimental import pallas as pl
from jax.experimental.pallas import tpu as pltpu

NUM_GROUPS = 8
NUM_CODES = 1024
GROUP_DIM = 32
COMMITMENT_COST = 0.25

TILE_N = 512


def _vq_kernel(x_ref, xsq_ref, embt_ref, esq_ref, q_ref, idx_ref, sse_ref):
    tn = x_ref.shape[0]
    psum = jnp.float32(0.0)
    idx_cols = []
    q_cols = []
    for g in range(NUM_GROUPS):
        xg = x_ref[:, g * GROUP_DIM:(g + 1) * GROUP_DIM]
        embt_g = embt_ref[g]
        dot = jax.lax.dot_general(
            xg, embt_g, (((1,), (0,)), ((), ())),
            preferred_element_type=jnp.float32)
        a = xsq_ref[:, g:g + 1] + esq_ref[g][None, :]
        dist = a - 2.0 * dot
        mind = jnp.min(dist, axis=1)
        psum = psum + jnp.sum(mind)
        iota = jax.lax.broadcasted_iota(jnp.int32, (tn, NUM_CODES), 1)
        idx = jnp.min(jnp.where(dist == mind[:, None], iota, NUM_CODES),
                      axis=1)
        onehot = (iota == idx[:, None]).astype(jnp.float32)
        qg = jax.lax.dot_general(
            onehot, embt_g, (((1,), (1,)), ((), ())),
            preferred_element_type=jnp.float32,
            precision=jax.lax.Precision.HIGHEST)
        idx_cols.append(idx)
        q_cols.append(qg)
    q_ref[...] = jnp.concatenate(q_cols, axis=1)
    idx_ref[...] = jnp.stack(idx_cols, axis=1)

    @pl.when(pl.program_id(0) == 0)
    def _init():
        sse_ref[0, 0] = jnp.float32(0.0)

    sse_ref[0, 0] += psum


@functools.partial(jax.jit, static_argnames=("interpret",))
def kernel(inputs, embedding, interpret=False):
    n = inputs.shape[0]
    x3 = inputs.reshape(n, NUM_GROUPS, GROUP_DIM)
    inputs_sq = jnp.sum(x3 ** 2, axis=2)
    embed_sq = jnp.sum(embedding ** 2, axis=2)
    emb_t = jnp.transpose(embedding, (0, 2, 1))

    grid = (n // TILE_N,)
    q, idx, sse = pl.pallas_call(
        _vq_kernel,
        grid=grid,
        in_specs=[
            pl.BlockSpec((TILE_N, NUM_GROUPS * GROUP_DIM), lambda i: (i, 0)),
            pl.BlockSpec((TILE_N, NUM_GROUPS), lambda i: (i, 0)),
            pl.BlockSpec((NUM_GROUPS, GROUP_DIM, NUM_CODES),
                         lambda i: (0, 0, 0)),
            pl.BlockSpec((NUM_GROUPS, NUM_CODES), lambda i: (0, 0)),
        ],
        out_specs=[
            pl.BlockSpec((TILE_N, NUM_GROUPS * GROUP_DIM), lambda i: (i, 0)),
            pl.BlockSpec((TILE_N, NUM_GROUPS), lambda i: (i, 0)),
            pl.BlockSpec((1, 1), lambda i: (0, 0),
                         memory_space=pltpu.SMEM),
        ],
        out_shape=[
            jax.ShapeDtypeStruct((n, NUM_GROUPS * GROUP_DIM), jnp.float32),
            jax.ShapeDtypeStruct((n, NUM_GROUPS), jnp.int32),
            jax.ShapeDtypeStruct((1, 1), jnp.float32),
        ],
        interpret=interpret,
    )(inputs, inputs_sq, emb_t, embed_sq)

    total = jnp.float32(n * NUM_GROUPS * GROUP_DIM)
    codebook_loss = sse[0, 0] / total
    commit_loss = codebook_loss
    vq_loss = codebook_loss + COMMITMENT_COST * commit_loss
    indices = idx.astype(jnp.int64)
    return (q, indices, vq_loss, codebook_loss, commit_loss)


# speedup vs baseline: 4.5642x; 1.2880x over previous
import functools

import jax
import jax.numpy as jnp
from jax.experimental import pallas as pl
from jax.experimental.pallas import tpu as pltpu

NUM_GROUPS = 8
NUM_CODES = 1024
GROUP_DIM = 32
COMMITMENT_COST = 0.25

TILE_N = 512


def _vq_kernel(x_ref, xsq_ref, embt_ref, esq_ref, q_ref, idx_ref, sse_ref):
    tn = x_ref.shape[0]
    psum = jnp.float32(0.0)
    idx_cols = []
    q_cols = []
    for g in range(NUM_GROUPS):
        xg = x_ref[:, g * GROUP_DIM:(g + 1) * GROUP_DIM]
        embt_g = embt_ref[g]
        dot = jax.lax.dot_general(
            xg, embt_g, (((1,), (0,)), ((), ())),
            preferred_element_type=jnp.float32)
        a = xsq_ref[:, g:g + 1] + esq_ref[g][None, :]
        dist = a - 2.0 * dot
        mind = jnp.min(dist, axis=1)
        psum = psum + jnp.sum(mind)
        iota = jax.lax.broadcasted_iota(jnp.int32, (tn, NUM_CODES), 1)
        idx = jnp.min(jnp.where(dist == mind[:, None], iota, NUM_CODES),
                      axis=1)
        onehot = (iota == idx[:, None]).astype(jnp.float32)
        qg = jax.lax.dot_general(
            onehot, embt_g, (((1,), (1,)), ((), ())),
            preferred_element_type=jnp.float32)
        idx_cols.append(idx)
        q_cols.append(qg)
    q_ref[...] = jnp.concatenate(q_cols, axis=1)
    idx_ref[...] = jnp.stack(idx_cols, axis=1)

    @pl.when(pl.program_id(0) == 0)
    def _init():
        sse_ref[0, 0] = jnp.float32(0.0)

    sse_ref[0, 0] += psum


@functools.partial(jax.jit, static_argnames=("interpret",))
def kernel(inputs, embedding, interpret=False):
    n = inputs.shape[0]
    x3 = inputs.reshape(n, NUM_GROUPS, GROUP_DIM)
    inputs_sq = jnp.sum(x3 ** 2, axis=2)
    embed_sq = jnp.sum(embedding ** 2, axis=2)
    emb_t = jnp.transpose(embedding, (0, 2, 1))

    grid = (n // TILE_N,)
    q, idx, sse = pl.pallas_call(
        _vq_kernel,
        grid=grid,
        in_specs=[
            pl.BlockSpec((TILE_N, NUM_GROUPS * GROUP_DIM), lambda i: (i, 0)),
            pl.BlockSpec((TILE_N, NUM_GROUPS), lambda i: (i, 0)),
            pl.BlockSpec((NUM_GROUPS, GROUP_DIM, NUM_CODES),
                         lambda i: (0, 0, 0)),
            pl.BlockSpec((NUM_GROUPS, NUM_CODES), lambda i: (0, 0)),
        ],
        out_specs=[
            pl.BlockSpec((TILE_N, NUM_GROUPS * GROUP_DIM), lambda i: (i, 0)),
            pl.BlockSpec((TILE_N, NUM_GROUPS), lambda i: (i, 0)),
            pl.BlockSpec((1, 1), lambda i: (0, 0),
                         memory_space=pltpu.SMEM),
        ],
        out_shape=[
            jax.ShapeDtypeStruct((n, NUM_GROUPS * GROUP_DIM), jnp.float32),
            jax.ShapeDtypeStruct((n, NUM_GROUPS), jnp.int32),
            jax.ShapeDtypeStruct((1, 1), jnp.float32),
        ],
        interpret=interpret,
    )(inputs, inputs_sq, emb_t, embed_sq)

    total = jnp.float32(n * NUM_GROUPS * GROUP_DIM)
    codebook_loss = sse[0, 0] / total
    commit_loss = codebook_loss
    vq_loss = codebook_loss + COMMITMENT_COST * commit_loss
    indices = idx.astype(jnp.int64)
    return (q, indices, vq_loss, codebook_loss, commit_loss)


# speedup vs baseline: 5.1172x; 1.1211x over previous
import functools

import jax
import jax.numpy as jnp
from jax.experimental import pallas as pl
from jax.experimental.pallas import tpu as pltpu

NUM_GROUPS = 8
NUM_CODES = 1024
GROUP_DIM = 32
COMMITMENT_COST = 0.25

TILE_N = 1024


def _vq_kernel(x_ref, xsq_ref, embt_ref, esq_ref, q_ref, idx_ref, sse_ref):
    tn = x_ref.shape[0]
    psum = jnp.float32(0.0)
    idx_cols = []
    q_cols = []
    for g in range(NUM_GROUPS):
        xg = x_ref[:, g * GROUP_DIM:(g + 1) * GROUP_DIM]
        embt_g = embt_ref[g]
        dot = jax.lax.dot_general(
            xg, embt_g, (((1,), (0,)), ((), ())),
            preferred_element_type=jnp.float32)
        a = xsq_ref[:, g:g + 1] + esq_ref[g][None, :]
        dist = a - 2.0 * dot
        mind = jnp.min(dist, axis=1)
        psum = psum + jnp.sum(mind)
        iota = jax.lax.broadcasted_iota(jnp.int32, (tn, NUM_CODES), 1)
        idx = jnp.min(jnp.where(dist == mind[:, None], iota, NUM_CODES),
                      axis=1)
        onehot = (iota == idx[:, None]).astype(jnp.float32)
        qg = jax.lax.dot_general(
            onehot, embt_g, (((1,), (1,)), ((), ())),
            preferred_element_type=jnp.float32)
        idx_cols.append(idx)
        q_cols.append(qg)
    q_ref[...] = jnp.concatenate(q_cols, axis=1)
    idx_ref[...] = jnp.stack(idx_cols, axis=1)

    @pl.when(pl.program_id(0) == 0)
    def _init():
        sse_ref[0, 0] = jnp.float32(0.0)

    sse_ref[0, 0] += psum


@functools.partial(jax.jit, static_argnames=("interpret",))
def kernel(inputs, embedding, interpret=False):
    n = inputs.shape[0]
    x3 = inputs.reshape(n, NUM_GROUPS, GROUP_DIM)
    inputs_sq = jnp.sum(x3 ** 2, axis=2)
    embed_sq = jnp.sum(embedding ** 2, axis=2)
    emb_t = jnp.transpose(embedding, (0, 2, 1))

    grid = (n // TILE_N,)
    q, idx, sse = pl.pallas_call(
        _vq_kernel,
        grid=grid,
        in_specs=[
            pl.BlockSpec((TILE_N, NUM_GROUPS * GROUP_DIM), lambda i: (i, 0)),
            pl.BlockSpec((TILE_N, NUM_GROUPS), lambda i: (i, 0)),
            pl.BlockSpec((NUM_GROUPS, GROUP_DIM, NUM_CODES),
                         lambda i: (0, 0, 0)),
            pl.BlockSpec((NUM_GROUPS, NUM_CODES), lambda i: (0, 0)),
        ],
        out_specs=[
            pl.BlockSpec((TILE_N, NUM_GROUPS * GROUP_DIM), lambda i: (i, 0)),
            pl.BlockSpec((TILE_N, NUM_GROUPS), lambda i: (i, 0)),
            pl.BlockSpec((1, 1), lambda i: (0, 0),
                         memory_space=pltpu.SMEM),
        ],
        out_shape=[
            jax.ShapeDtypeStruct((n, NUM_GROUPS * GROUP_DIM), jnp.float32),
            jax.ShapeDtypeStruct((n, NUM_GROUPS), jnp.int32),
            jax.ShapeDtypeStruct((1, 1), jnp.float32),
        ],
        interpret=interpret,
    )(inputs, inputs_sq, emb_t, embed_sq)

    total = jnp.float32(n * NUM_GROUPS * GROUP_DIM)
    codebook_loss = sse[0, 0] / total
    commit_loss = codebook_loss
    vq_loss = codebook_loss + COMMITMENT_COST * commit_loss
    indices = idx.astype(jnp.int64)
    return (q, indices, vq_loss, codebook_loss, commit_loss)
